# Initial kernel scaffold; baseline (speedup 1.0000x reference)
import jax
import jax.numpy as jnp
from jax.experimental import pallas as pl


def kernel(features, sparse, c_param, W_ac1, b_ac1, W_ca1, b_ca1, W_ac2, b_ac2, W_ca2, b_ca2, bn1_g, bn1_b, bn2_g, bn2_b, W_out):
    raise NotImplementedError("write your pallas kernel here")


# speedup vs baseline: 4.0776x; 4.0776x over previous
import functools

import jax
import jax.numpy as jnp
from jax.experimental import pallas as pl
from jax.experimental.pallas import tpu as pltpu

NN = 1024
BS = 4 * NN
F0 = 64
H = 128
NC = 6


def _norm_kernel(x_ref, out_ref):
    x = x_ref[...]
    xc = x - jnp.mean(x, axis=1, keepdims=True)
    out_ref[...] = xc * jax.lax.rsqrt(jnp.sum(xc * xc, axis=1, keepdims=True))


def _adj_kernel(thr_ref, xni_ref, xna_ref, xnb_ref, adj_ref, pr_ref, pc_ref):
    i = pl.program_id(0)
    j = pl.program_id(1)

    xn_i = xni_ref[...]
    g_ii = jax.lax.dot_general(xn_i, xn_i, (((1,), (1,)), ((), ())),
                               preferred_element_type=jnp.float32)
    rows = jax.lax.broadcasted_iota(jnp.int32, (NN, NN), 0)
    cols = jax.lax.broadcasted_iota(jnp.int32, (NN, NN), 1)
    eye = rows == cols
    mask_a = (jnp.abs(g_ii) > thr_ref[0, i]) & (~eye)

    p = jax.lax.dot_general(xna_ref[...], xnb_ref[...], (((1,), (1,)), ((), ())),
                            preferred_element_type=jnp.float32)
    th_c = thr_ref[0, 4 + jnp.maximum(i, j)]
    mask_c = jnp.where(i == j, eye.astype(jnp.bfloat16),
                       (jnp.abs(p) > th_c).astype(jnp.bfloat16))

    cnt = jax.lax.dot_general(mask_a.astype(jnp.bfloat16), mask_c,
                              (((1,), (0,)), ((), ())),
                              preferred_element_type=jnp.float32)
    ind = (cnt > 0.0).astype(jnp.float32)
    adj_ref[...] = ind.astype(jnp.bfloat16)
    ones = jnp.ones((NN, 1), jnp.float32)
    pr_ref[...] = jnp.sum(ind, axis=1, keepdims=True)[None]
    pc_ref[...] = jax.lax.dot_general(ind, ones, (((0,), (0,)), ((), ())),
                                      preferred_element_type=jnp.float32)[None]


def _degsum_kernel(pr_ref, pc_ref, degr_ref, degc_ref):
    degr_ref[...] = jnp.sum(pr_ref[...], axis=0)
    degc_ref[...] = jnp.sum(pc_ref[...], axis=0)


def _degnorm(deg):
    n = jnp.where(deg > 0, jax.lax.rsqrt(jnp.maximum(deg, 1.0)), 0.0)
    return deg, n


def _gin_block(adj_col, adj_row, x, x_d, n_c, n_r,
               degc_d, n_c_d, degr_d, n_r_d,
               w_ac, b_ac, w_ca, b_ca):
    u = (x * n_c).astype(jnp.bfloat16)
    v = (x * n_r).astype(jnp.bfloat16)
    agg_ac = jax.lax.dot_general(adj_col, u,
                                 (((0,), (0,)), ((), ())),
                                 preferred_element_type=jnp.float32)
    agg_ca = jax.lax.dot_general(adj_row, v,
                                 (((1,), (0,)), ((), ())),
                                 preferred_element_type=jnp.float32)
    agg_ac = agg_ac * n_c_d / jnp.maximum(degc_d, 1.0)
    agg_ca = agg_ca * n_r_d / jnp.maximum(degr_d, 1.0)
    z_ac = jax.nn.relu(
        jnp.dot(x_d + agg_ac, w_ac, preferred_element_type=jnp.float32) + b_ac)
    z_ca = jax.nn.relu(
        jnp.dot(x_d + agg_ca, w_ca, preferred_element_type=jnp.float32) + b_ca)
    return jnp.concatenate([z_ac, z_ca], axis=1)


def _l1_kernel(adj_col_ref, adj_row_ref, x_ref, xd_ref, pr_ref, pc_ref,
               prd_ref, pcd_ref,
               wac_ref, bac_ref, wca_ref, bca_ref,
               h_ref, ss_ref, sq_ref):
    degr, n_r = _degnorm(pr_ref[...])
    degc, n_c = _degnorm(pc_ref[...])
    degr_d, n_r_d = _degnorm(prd_ref[...])
    degc_d, n_c_d = _degnorm(pcd_ref[...])
    h_d = _gin_block(adj_col_ref[...], adj_row_ref[...], x_ref[...],
                     xd_ref[...], n_c, n_r, degc_d, n_c_d, degr_d, n_r_d,
                     wac_ref[...], bac_ref[...], wca_ref[...], bca_ref[...])
    h_ref[...] = h_d
    ss_ref[...] = jnp.sum(h_d, axis=0, keepdims=True)[None]
    sq_ref[...] = jnp.sum(h_d * h_d, axis=0, keepdims=True)[None]


def _l2_kernel(adj_col_ref, adj_row_ref, h1_ref, h1d_ref, ss_ref, sq_ref,
               g_ref, b_ref, pr_ref, pc_ref, prd_ref, pcd_ref,
               wac_ref, bac_ref, wca_ref, bca_ref,
               h_ref, ss2_ref, sq2_ref):
    mu = jnp.sum(ss_ref[...][:, 0, :], axis=0, keepdims=True) / BS
    msq = jnp.sum(sq_ref[...][:, 0, :], axis=0, keepdims=True) / BS
    var = msq - mu * mu
    scale = jax.lax.rsqrt(var + 1e-5) * g_ref[...]
    bias = b_ref[...]
    x = (h1_ref[...] - mu) * scale + bias
    x_d = (h1d_ref[...] - mu) * scale + bias
    degr, n_r = _degnorm(pr_ref[...])
    degc, n_c = _degnorm(pc_ref[...])
    degr_d, n_r_d = _degnorm(prd_ref[...])
    degc_d, n_c_d = _degnorm(pcd_ref[...])
    h_d = _gin_block(adj_col_ref[...], adj_row_ref[...], x, x_d,
                     n_c, n_r, degc_d, n_c_d, degr_d, n_r_d,
                     wac_ref[...], bac_ref[...], wca_ref[...], bca_ref[...])
    h_ref[...] = h_d
    ss2_ref[...] = jnp.sum(h_d, axis=0, keepdims=True)[None]
    sq2_ref[...] = jnp.sum(h_d * h_d, axis=0, keepdims=True)[None]


def _out_kernel(c_ref, h2_ref, ss_ref, sq_ref, g_ref, b_ref, wout_ref,
                out_ref):
    mu = jnp.sum(ss_ref[...][:, 0, :], axis=0, keepdims=True) / BS
    msq = jnp.sum(sq_ref[...][:, 0, :], axis=0, keepdims=True) / BS
    var = msq - mu * mu
    scale = jax.lax.rsqrt(var + 1e-5) * g_ref[...]
    h = (h2_ref[...] - mu) * scale + b_ref[...]
    y = jnp.dot(h, wout_ref[...], preferred_element_type=jnp.float32)
    c0 = c_ref[0, 0]
    c1 = c_ref[0, 1]
    c2 = c_ref[0, 2]
    c3 = c_ref[0, 3]
    m = jnp.maximum(jnp.maximum(c0, c1), jnp.maximum(c2, c3))
    e0 = jnp.exp(c0 - m)
    e1 = jnp.exp(c1 - m)
    e2 = jnp.exp(c2 - m)
    e3 = jnp.exp(c3 - m)
    den = e0 + e1 + e2 + e3
    out_ref[...] = (y[0 * NN:1 * NN] * (e0 / den) +
                    y[1 * NN:2 * NN] * (e1 / den) +
                    y[2 * NN:3 * NN] * (e2 / den) +
                    y[3 * NN:4 * NN] * (e3 / den))


@functools.partial(jax.jit, static_argnames=())
def kernel(features, sparse, c_param, W_ac1, b_ac1, W_ca1, b_ca1,
           W_ac2, b_ac2, W_ca2, b_ca2, bn1_g, bn1_b, bn2_g, bn2_b, W_out):
    f32 = jnp.float32

    xn = pl.pallas_call(
        _norm_kernel,
        out_shape=jax.ShapeDtypeStruct((BS, F0), f32),
    )(features)

    sig = jax.nn.sigmoid(sparse[:, 0])
    thr = jnp.stack([sig[1], sig[5], sig[8], sig[10],
                     jnp.float32(0.0), sig[4], sig[7], sig[9]])[None, :]

    adj, parts_r, parts_c = pl.pallas_call(
        _adj_kernel,
        grid=(4, 4),
        in_specs=[
            pl.BlockSpec(memory_space=pltpu.SMEM),
            pl.BlockSpec((NN, F0), lambda i, j: (i, 0)),
            pl.BlockSpec((NN, F0),
                         lambda i, j: (jnp.where(i > j, i - 1, 3), 0)),
            pl.BlockSpec((NN, F0),
                         lambda i, j: (jnp.where(i > j, 3,
                                                 jnp.maximum(j - 1, 0)), 0)),
        ],
        out_specs=[
            pl.BlockSpec((NN, NN), lambda i, j: (i, j)),
            pl.BlockSpec((1, NN, 1), lambda i, j: (j, i, 0)),
            pl.BlockSpec((1, NN, 1), lambda i, j: (i, j, 0)),
        ],
        out_shape=[
            jax.ShapeDtypeStruct((BS, BS), jnp.bfloat16),
            jax.ShapeDtypeStruct((4, BS, 1), f32),
            jax.ShapeDtypeStruct((4, BS, 1), f32),
        ],
    )(thr, xn, xn, xn)

    degr, degc = pl.pallas_call(
        _degsum_kernel,
        out_shape=[jax.ShapeDtypeStruct((BS, 1), f32),
                   jax.ShapeDtypeStruct((BS, 1), f32)],
    )(parts_r, parts_c)

    def layer_specs(feat):
        return [
            pl.BlockSpec((BS, NN), lambda d: (0, d)),
            pl.BlockSpec((NN, BS), lambda d: (d, 0)),
        ]

    b2 = lambda a: a[None, :]

    h1, ss1, sq1 = pl.pallas_call(
        _l1_kernel,
        grid=(4,),
        in_specs=layer_specs(F0) + [
            pl.BlockSpec((BS, F0), lambda d: (0, 0)),
            pl.BlockSpec((NN, F0), lambda d: (d, 0)),
            pl.BlockSpec((BS, 1), lambda d: (0, 0)),
            pl.BlockSpec((BS, 1), lambda d: (0, 0)),
            pl.BlockSpec((NN, 1), lambda d: (d, 0)),
            pl.BlockSpec((NN, 1), lambda d: (d, 0)),
            pl.BlockSpec((F0, H), lambda d: (0, 0)),
            pl.BlockSpec((1, H), lambda d: (0, 0)),
            pl.BlockSpec((F0, H), lambda d: (0, 0)),
            pl.BlockSpec((1, H), lambda d: (0, 0)),
        ],
        out_specs=[
            pl.BlockSpec((NN, 2 * H), lambda d: (d, 0)),
            pl.BlockSpec((1, 1, 2 * H), lambda d: (d, 0, 0)),
            pl.BlockSpec((1, 1, 2 * H), lambda d: (d, 0, 0)),
        ],
        out_shape=[
            jax.ShapeDtypeStruct((BS, 2 * H), f32),
            jax.ShapeDtypeStruct((4, 1, 2 * H), f32),
            jax.ShapeDtypeStruct((4, 1, 2 * H), f32),
        ],
    )(adj, adj, features, features, degr, degc, degr, degc,
      W_ac1, b2(b_ac1), W_ca1, b2(b_ca1))

    h2, ss2, sq2 = pl.pallas_call(
        _l2_kernel,
        grid=(4,),
        in_specs=layer_specs(2 * H) + [
            pl.BlockSpec((BS, 2 * H), lambda d: (0, 0)),
            pl.BlockSpec((NN, 2 * H), lambda d: (d, 0)),
            pl.BlockSpec((4, 1, 2 * H), lambda d: (0, 0, 0)),
            pl.BlockSpec((4, 1, 2 * H), lambda d: (0, 0, 0)),
            pl.BlockSpec((1, 2 * H), lambda d: (0, 0)),
            pl.BlockSpec((1, 2 * H), lambda d: (0, 0)),
            pl.BlockSpec((BS, 1), lambda d: (0, 0)),
            pl.BlockSpec((BS, 1), lambda d: (0, 0)),
            pl.BlockSpec((NN, 1), lambda d: (d, 0)),
            pl.BlockSpec((NN, 1), lambda d: (d, 0)),
            pl.BlockSpec((2 * H, H), lambda d: (0, 0)),
            pl.BlockSpec((1, H), lambda d: (0, 0)),
            pl.BlockSpec((2 * H, H), lambda d: (0, 0)),
            pl.BlockSpec((1, H), lambda d: (0, 0)),
        ],
        out_specs=[
            pl.BlockSpec((NN, 2 * H), lambda d: (d, 0)),
            pl.BlockSpec((1, 1, 2 * H), lambda d: (d, 0, 0)),
            pl.BlockSpec((1, 1, 2 * H), lambda d: (d, 0, 0)),
        ],
        out_shape=[
            jax.ShapeDtypeStruct((BS, 2 * H), f32),
            jax.ShapeDtypeStruct((4, 1, 2 * H), f32),
            jax.ShapeDtypeStruct((4, 1, 2 * H), f32),
        ],
    )(adj, adj, h1, h1, ss1, sq1, b2(bn1_g), b2(bn1_b),
      degr, degc, degr, degc,
      W_ac2, b2(b_ac2), W_ca2, b2(b_ca2))

    out = pl.pallas_call(
        _out_kernel,
        in_specs=[
            pl.BlockSpec(memory_space=pltpu.SMEM),
            pl.BlockSpec((BS, 2 * H), lambda: (0, 0)),
            pl.BlockSpec((4, 1, 2 * H), lambda: (0, 0, 0)),
            pl.BlockSpec((4, 1, 2 * H), lambda: (0, 0, 0)),
            pl.BlockSpec((1, 2 * H), lambda: (0, 0)),
            pl.BlockSpec((1, 2 * H), lambda: (0, 0)),
            pl.BlockSpec((2 * H, NC), lambda: (0, 0)),
        ],
        out_shape=jax.ShapeDtypeStruct((NN, NC), f32),
    )(c_param, h2, ss2, sq2, b2(bn2_g), b2(bn2_b), W_out)

    return out
